# baseline (speedup 1.0000x reference)
import jax
import jax.numpy as jnp
from jax.experimental import pallas as pl

B_, N_ = 4, 4096
IN_C, CGEO, N1, N2, KFP = 3, 256, 512, 128, 3
RADII1, NS1 = (0.1, 0.2, 0.4), (16, 32, 128)
RADII2, NS2 = (0.2, 0.4, 0.8), (32, 64, 128)


def _dense_nb_body(x_ref, w_ref, o_ref):
    o_ref[...] = jnp.dot(
        x_ref[...],
        w_ref[...],
        preferred_element_type=jnp.float32,
    )


def _dense_nb(x, W, tile_rows=512):
    R, Cin = x.shape
    Cout = W.shape[0]
    Rp = (R + tile_rows - 1) // tile_rows * tile_rows
    if Rp != R:
        x = jnp.pad(x, ((0, Rp - R), (0, 0)))
    out = pl.pallas_call(
        _dense_nb_body,
        grid=(Rp // tile_rows,),
        in_specs=[
            pl.BlockSpec((tile_rows, Cin), lambda i: (i, 0)),
            pl.BlockSpec((Cin, Cout), lambda i: (0, 0)),
        ],
        out_specs=pl.BlockSpec((tile_rows, Cout), lambda i: (i, 0)),
        out_shape=jax.ShapeDtypeStruct((Rp, Cout), jnp.float32),
    )(x, W.T)
    return out[:R]


def _apply_mlp(layers, h, mask=None, pallas=True):
    red = tuple(range(h.ndim - 1))
    for lyr in layers:
        if pallas:
            shp = h.shape
            h2 = _dense_nb(h.reshape(-1, shp[-1]), lyr["W"])
            h = h2.reshape(shp[:-1] + (lyr["W"].shape[0],)) + lyr["b"]
        else:
            h = h @ lyr["W"].T + lyr["b"]
        if mask is None:
            mean = h.mean(axis=red)
            var = ((h - mean) ** 2).mean(axis=red)
        else:
            m = mask[..., None].astype(h.dtype)
            cnt = jnp.maximum(mask.astype(h.dtype).sum(), 1.0)
            mean = (h * m).sum(axis=red) / cnt
            var = (((h - mean) ** 2) * m).sum(axis=red) / cnt
        h = (h - mean) / jnp.sqrt(var + 1e-5) * lyr["gamma"] + lyr["beta"]
        h = jax.nn.relu(h)
    return h


def _fps(pos_b, n_samples):
    dists = jnp.full((pos_b.shape[0],), jnp.inf, dtype=pos_b.dtype)
    idxs = jnp.zeros((n_samples,), dtype=jnp.int32)

    def body(i, carry):
        idxs, dists = carry
        d = jnp.sum((pos_b - pos_b[idxs[i - 1]]) ** 2, axis=1)
        dists = jnp.minimum(dists, d)
        return (idxs.at[i].set(jnp.argmax(dists).astype(jnp.int32)), dists)

    idxs, _ = jax.lax.fori_loop(1, n_samples, body, (idxs, dists))
    return idxs


def _gather(a, idx):
    return jax.vmap(lambda ab, ib: ab[ib])(a, idx)


def _msg_sa(x_flat, pos, pos_s, radii, nsamples, conv_params, pallas=True):
    B, N, _ = pos.shape
    M = pos_s.shape[1]
    C = x_flat.shape[1]
    x = x_flat.reshape(B, N, C)
    d2 = jnp.sum((pos_s[:, :, None, :] - pos[:, None, :, :]) ** 2, axis=-1)
    pos_flat = pos.reshape(B * N, 3)
    pos_s_flat = pos_s.reshape(B * M, 3)
    x_self = x_flat[: B * M]
    rel_self = pos_flat[: B * M] - pos_s_flat
    msg_self = jnp.concatenate([x_self, rel_self], axis=1)[:, None, :]
    outs = []
    for r, k, layers in zip(radii, nsamples, conv_params):
        neg, nidx = jax.lax.top_k(-d2, k)
        mask = ((-neg) <= r * r).reshape(B * M, k)
        x_j = _gather(x, nidx).reshape(B * M, k, C)
        pos_j = _gather(pos, nidx)
        rel = (pos_j - pos_s[:, :, None, :]).reshape(B * M, k, 3)
        msg = jnp.concatenate([x_j, rel], axis=2)
        msgs = jnp.concatenate([msg, msg_self], axis=1)
        mfull = jnp.concatenate([mask, jnp.ones((B * M, 1), bool)], axis=1)
        h = _apply_mlp(layers, msgs, mfull, pallas=pallas)
        out = jnp.max(jnp.where(mfull[..., None], h, -jnp.inf), axis=1)
        outs.append(out)
    return jnp.concatenate(outs, axis=1)


def _knn_interp(x, pos_x, pos_y, k):
    d2 = jnp.sum((pos_y[:, :, None, :] - pos_x[:, None, :, :]) ** 2, axis=-1)
    neg, idx = jax.lax.top_k(-d2, k)
    w = 1.0 / jnp.maximum(-neg, 1e-16)
    feats = _gather(x, idx)
    return (feats * w[..., None]).sum(axis=2) / w.sum(axis=2, keepdims=True)


@jax.jit
def _forward(pts, params):
    B, N, _ = pts.shape
    pos = pts
    x0 = pts.reshape(B * N, 3)
    idx1 = jax.vmap(lambda p: _fps(p, N1))(pos)
    pos1 = _gather(pos, idx1)
    x1 = _msg_sa(x0, pos, pos1, RADII1, NS1, params["sa1"], pallas=False)
    idx2 = jax.vmap(lambda p: _fps(p, N2))(pos1)
    pos2 = _gather(pos1, idx2)
    x2 = _msg_sa(x1, pos1, pos2, RADII2, NS2, params["sa2"])
    C2 = x2.shape[1]
    g = _apply_mlp(params["glob"], x2.reshape(B, N2, C2).max(axis=1))
    x1_up = _knn_interp(x2.reshape(B, N2, C2), pos2, pos1, KFP).reshape(B * N1, C2)
    x1_fp = _apply_mlp(params["fp1"], jnp.concatenate([x1_up, x1], axis=1))
    x0_up = _knn_interp(x1_fp.reshape(B, N1, 256), pos1, pos, KFP).reshape(B * N, 256)
    F = _apply_mlp(params["fp0"], jnp.concatenate([x0_up, x0], axis=1))
    return F.reshape(B, N, CGEO), g


def kernel(pts, params):
    return _forward(pts, params)


# speedup vs baseline: 1.0013x; 1.0013x over previous
import jax
import jax.numpy as jnp
from jax.experimental import pallas as pl

B_, N_ = 4, 4096
IN_C, CGEO, N1, N2, KFP = 3, 256, 512, 128, 3
RADII1, NS1 = (0.1, 0.2, 0.4), (16, 32, 128)
RADII2, NS2 = (0.2, 0.4, 0.8), (32, 64, 128)


def _dense_nb_body(x_ref, w_ref, o_ref):
    o_ref[...] = jnp.dot(
        x_ref[...],
        w_ref[...],
        preferred_element_type=jnp.float32,
    )


def _dense_nb(x, W, tile_rows=1024):
    R, Cin = x.shape
    Cout = W.shape[0]
    Rp = (R + tile_rows - 1) // tile_rows * tile_rows
    if Rp != R:
        x = jnp.pad(x, ((0, Rp - R), (0, 0)))
    out = pl.pallas_call(
        _dense_nb_body,
        grid=(Rp // tile_rows,),
        in_specs=[
            pl.BlockSpec((tile_rows, Cin), lambda i: (i, 0)),
            pl.BlockSpec((Cin, Cout), lambda i: (0, 0)),
        ],
        out_specs=pl.BlockSpec((tile_rows, Cout), lambda i: (i, 0)),
        out_shape=jax.ShapeDtypeStruct((Rp, Cout), jnp.float32),
    )(x, W.T)
    return out[:R]


def _apply_mlp(layers, h, mask=None, pallas=True):
    red = tuple(range(h.ndim - 1))
    for lyr in layers:
        if pallas:
            shp = h.shape
            h2 = _dense_nb(h.reshape(-1, shp[-1]), lyr["W"])
            h = h2.reshape(shp[:-1] + (lyr["W"].shape[0],)) + lyr["b"]
        else:
            h = h @ lyr["W"].T + lyr["b"]
        if mask is None:
            mean = h.mean(axis=red)
            var = ((h - mean) ** 2).mean(axis=red)
        else:
            m = mask[..., None].astype(h.dtype)
            cnt = jnp.maximum(mask.astype(h.dtype).sum(), 1.0)
            mean = (h * m).sum(axis=red) / cnt
            var = (((h - mean) ** 2) * m).sum(axis=red) / cnt
        h = (h - mean) / jnp.sqrt(var + 1e-5) * lyr["gamma"] + lyr["beta"]
        h = jax.nn.relu(h)
    return h


def _fps(pos_b, n_samples):
    dists = jnp.full((pos_b.shape[0],), jnp.inf, dtype=pos_b.dtype)
    idxs = jnp.zeros((n_samples,), dtype=jnp.int32)

    def body(i, carry):
        idxs, dists = carry
        d = jnp.sum((pos_b - pos_b[idxs[i - 1]]) ** 2, axis=1)
        dists = jnp.minimum(dists, d)
        return (idxs.at[i].set(jnp.argmax(dists).astype(jnp.int32)), dists)

    idxs, _ = jax.lax.fori_loop(1, n_samples, body, (idxs, dists))
    return idxs


def _gather(a, idx):
    return jax.vmap(lambda ab, ib: ab[ib])(a, idx)


def _msg_sa(x_flat, pos, pos_s, radii, nsamples, conv_params, pallas=True):
    B, N, _ = pos.shape
    M = pos_s.shape[1]
    C = x_flat.shape[1]
    x = x_flat.reshape(B, N, C)
    d2 = jnp.sum((pos_s[:, :, None, :] - pos[:, None, :, :]) ** 2, axis=-1)
    pos_flat = pos.reshape(B * N, 3)
    pos_s_flat = pos_s.reshape(B * M, 3)
    x_self = x_flat[: B * M]
    rel_self = pos_flat[: B * M] - pos_s_flat
    msg_self = jnp.concatenate([x_self, rel_self], axis=1)[:, None, :]
    outs = []
    for r, k, layers in zip(radii, nsamples, conv_params):
        neg, nidx = jax.lax.top_k(-d2, k)
        mask = ((-neg) <= r * r).reshape(B * M, k)
        x_j = _gather(x, nidx).reshape(B * M, k, C)
        pos_j = _gather(pos, nidx)
        rel = (pos_j - pos_s[:, :, None, :]).reshape(B * M, k, 3)
        msg = jnp.concatenate([x_j, rel], axis=2)
        msgs = jnp.concatenate([msg, msg_self], axis=1)
        mfull = jnp.concatenate([mask, jnp.ones((B * M, 1), bool)], axis=1)
        h = _apply_mlp(layers, msgs, mfull, pallas=pallas)
        out = jnp.max(jnp.where(mfull[..., None], h, -jnp.inf), axis=1)
        outs.append(out)
    return jnp.concatenate(outs, axis=1)


def _knn_interp(x, pos_x, pos_y, k):
    d2 = jnp.sum((pos_y[:, :, None, :] - pos_x[:, None, :, :]) ** 2, axis=-1)
    neg, idx = jax.lax.top_k(-d2, k)
    w = 1.0 / jnp.maximum(-neg, 1e-16)
    feats = _gather(x, idx)
    return (feats * w[..., None]).sum(axis=2) / w.sum(axis=2, keepdims=True)


@jax.jit
def _forward(pts, params):
    B, N, _ = pts.shape
    pos = pts
    x0 = pts.reshape(B * N, 3)
    idx1 = jax.vmap(lambda p: _fps(p, N1))(pos)
    pos1 = _gather(pos, idx1)
    x1 = _msg_sa(x0, pos, pos1, RADII1, NS1, params["sa1"], pallas=False)
    idx2 = jax.vmap(lambda p: _fps(p, N2))(pos1)
    pos2 = _gather(pos1, idx2)
    x2 = _msg_sa(x1, pos1, pos2, RADII2, NS2, params["sa2"])
    C2 = x2.shape[1]
    g = _apply_mlp(params["glob"], x2.reshape(B, N2, C2).max(axis=1))
    x1_up = _knn_interp(x2.reshape(B, N2, C2), pos2, pos1, KFP).reshape(B * N1, C2)
    x1_fp = _apply_mlp(params["fp1"], jnp.concatenate([x1_up, x1], axis=1))
    x0_up = _knn_interp(x1_fp.reshape(B, N1, 256), pos1, pos, KFP).reshape(B * N, 256)
    F = _apply_mlp(params["fp0"], jnp.concatenate([x0_up, x0], axis=1))
    return F.reshape(B, N, CGEO), g


def kernel(pts, params):
    return _forward(pts, params)
